# speedup vs baseline: 1.8320x; 1.8320x over previous
import jax
import jax.numpy as jnp
from jax.experimental import pallas as pl
from jax.experimental.pallas import tpu as pltpu

_EPS = 1e-5
_CB = 16


def _cin_kernel(styles_ref, x_ref, g_ref, b_ref, o_ref):
    del styles_ref
    x = x_ref[...]
    mean = jnp.mean(x, axis=(2, 3), keepdims=True)
    xc = x - mean
    var = jnp.mean(xc * xc, axis=(2, 3), keepdims=True)
    scale = jax.lax.rsqrt(var + _EPS) * g_ref[...]
    o_ref[...] = xc * scale + b_ref[...]


def kernel(x, styles, gamma, beta):
    B, C, H, W = x.shape
    S = gamma.shape[0]
    styles = styles.astype(jnp.int32)
    g4 = gamma.reshape(S, C, 1, 1)
    b4 = beta.reshape(S, C, 1, 1)

    grid = (B, C // _CB)
    grid_spec = pltpu.PrefetchScalarGridSpec(
        num_scalar_prefetch=1,
        grid=grid,
        in_specs=[
            pl.BlockSpec((1, _CB, H, W), lambda i, j, s: (i, j, 0, 0)),
            pl.BlockSpec((1, _CB, 1, 1), lambda i, j, s: (s[i], j, 0, 0)),
            pl.BlockSpec((1, _CB, 1, 1), lambda i, j, s: (s[i], j, 0, 0)),
        ],
        out_specs=pl.BlockSpec((1, _CB, H, W), lambda i, j, s: (i, j, 0, 0)),
    )
    return pl.pallas_call(
        _cin_kernel,
        out_shape=jax.ShapeDtypeStruct((B, C, H, W), x.dtype),
        grid_spec=grid_spec,
        compiler_params=pltpu.CompilerParams(
            dimension_semantics=("parallel", "parallel"),
        ),
        name="conditional_instance_norm",
    )(styles, x, g4, b4)


# speedup vs baseline: 1.8858x; 1.0293x over previous
import jax
import jax.numpy as jnp
from jax.experimental import pallas as pl
from jax.experimental.pallas import tpu as pltpu

_EPS = 1e-5
_CB = 32


def _cin_kernel(styles_ref, x_ref, g_ref, b_ref, o_ref):
    del styles_ref
    x = x_ref[...]
    mean = jnp.mean(x, axis=(2, 3), keepdims=True)
    xc = x - mean
    var = jnp.mean(xc * xc, axis=(2, 3), keepdims=True)
    scale = jax.lax.rsqrt(var + _EPS) * g_ref[...]
    o_ref[...] = xc * scale + b_ref[...]


def kernel(x, styles, gamma, beta):
    B, C, H, W = x.shape
    S = gamma.shape[0]
    styles = styles.astype(jnp.int32)
    g4 = gamma.reshape(S, C, 1, 1)
    b4 = beta.reshape(S, C, 1, 1)

    grid = (B, C // _CB)
    grid_spec = pltpu.PrefetchScalarGridSpec(
        num_scalar_prefetch=1,
        grid=grid,
        in_specs=[
            pl.BlockSpec((1, _CB, H, W), lambda i, j, s: (i, j, 0, 0)),
            pl.BlockSpec((1, _CB, 1, 1), lambda i, j, s: (s[i], j, 0, 0)),
            pl.BlockSpec((1, _CB, 1, 1), lambda i, j, s: (s[i], j, 0, 0)),
        ],
        out_specs=pl.BlockSpec((1, _CB, H, W), lambda i, j, s: (i, j, 0, 0)),
    )
    return pl.pallas_call(
        _cin_kernel,
        out_shape=jax.ShapeDtypeStruct((B, C, H, W), x.dtype),
        grid_spec=grid_spec,
        compiler_params=pltpu.CompilerParams(
            dimension_semantics=("parallel", "parallel"),
        ),
        name="conditional_instance_norm",
    )(styles, x, g4, b4)
